# Initial kernel scaffold; baseline (speedup 1.0000x reference)
import functools

import jax
import jax.numpy as jnp
from jax import lax
from jax.experimental import pallas as pl
from jax.experimental.pallas import tpu as pltpu
from jax.experimental.pallas import tpu_sc as plsc

_D = 32


@functools.cache
def _make_gather(B: int, C: int):
    info = plsc.get_sparse_core_info()
    nc, ns = info.num_cores, info.num_subcores
    nw = nc * ns
    b_per_w = B // nw
    n_chunks = b_per_w // C
    assert b_per_w % C == 0 and C % 8 == 0

    mesh = plsc.VectorSubcoreMesh(core_axis_name="c", subcore_axis_name="s")

    @functools.partial(
        pl.kernel,
        mesh=mesh,
        out_type=jax.ShapeDtypeStruct((B, _D), jnp.float32),
        scratch_types=[
            pltpu.VMEM((C,), jnp.int32),
            pltpu.VMEM((C, _D), jnp.float32),
            pltpu.SemaphoreType.DMA,
        ],
    )
    def gather_kernel(idx_hbm, table_hbm, out_hbm, idx_v, rows_v, sem):
        wid = lax.axis_index("s") * nc + lax.axis_index("c")
        base = wid * b_per_w

        def body(i, carry):
            off = base + i * C
            pltpu.sync_copy(idx_hbm.at[pl.ds(off, C)], idx_v)
            pltpu.async_copy(table_hbm.at[idx_v], rows_v, sem).wait()
            pltpu.sync_copy(rows_v, out_hbm.at[pl.ds(off, C)])
            return carry

        lax.fori_loop(0, n_chunks, body, 0)

    return gather_kernel


def kernel(token_ids, weight):
    s0, s1 = token_ids.shape
    B = s0 * s1
    idx = token_ids.reshape(B).astype(jnp.int32)
    out = _make_gather(B, 1024)(idx, weight)
    return out.reshape(s0, s1, _D)


# speedup vs baseline: 4.8110x; 4.8110x over previous
import functools

import jax
import jax.numpy as jnp
from jax import lax
from jax.experimental import pallas as pl
from jax.experimental.pallas import tpu as pltpu
from jax.experimental.pallas import tpu_sc as plsc

_D = 32


@functools.cache
def _make_gather(B: int, C: int):
    info = plsc.get_sparse_core_info()
    nc, ns = info.num_cores, info.num_subcores
    nw = nc * ns
    b_per_w = B // nw
    n_chunks = b_per_w // C
    assert b_per_w % C == 0 and C % 8 == 0

    mesh = plsc.VectorSubcoreMesh(core_axis_name="c", subcore_axis_name="s")

    @functools.partial(
        pl.kernel,
        mesh=mesh,
        out_type=jax.ShapeDtypeStruct((B, _D), jnp.float32),
        compiler_params=pltpu.CompilerParams(use_tc_tiling_on_sc=False),
        scratch_types=[
            pltpu.VMEM((C,), jnp.int32),
            pltpu.VMEM((C, _D), jnp.float32),
            pltpu.SemaphoreType.DMA,
        ],
    )
    def gather_kernel(idx_hbm, table_hbm, out_hbm, idx_v, rows_v, sem):
        wid = lax.axis_index("s") * nc + lax.axis_index("c")
        base = wid * b_per_w

        def body(i, carry):
            off = base + i * C
            pltpu.sync_copy(idx_hbm.at[pl.ds(off, C)], idx_v)
            pltpu.async_copy(table_hbm.at[idx_v], rows_v, sem).wait()
            pltpu.sync_copy(rows_v, out_hbm.at[pl.ds(off, C)])
            return carry

        lax.fori_loop(0, n_chunks, body, 0)

    return gather_kernel


def kernel(token_ids, weight):
    s0, s1 = token_ids.shape
    B = s0 * s1
    idx = token_ids.reshape(B).astype(jnp.int32)
    out = _make_gather(B, 1024)(idx, weight)
    return out.reshape(s0, s1, _D)


# speedup vs baseline: 5.0243x; 1.0443x over previous
import functools

import jax
import jax.numpy as jnp
from jax import lax
from jax.experimental import pallas as pl
from jax.experimental.pallas import tpu as pltpu
from jax.experimental.pallas import tpu_sc as plsc

_D = 32
_NBUF = 2


@functools.cache
def _make_gather(B: int, C: int):
    info = plsc.get_sparse_core_info()
    nc, ns = info.num_cores, info.num_subcores
    nw = nc * ns
    b_per_w = B // nw
    n_chunks = b_per_w // C
    assert b_per_w % C == 0 and C % 8 == 0 and n_chunks % _NBUF == 0

    mesh = plsc.VectorSubcoreMesh(core_axis_name="c", subcore_axis_name="s")

    @functools.partial(
        pl.kernel,
        mesh=mesh,
        out_type=jax.ShapeDtypeStruct((B, _D), jnp.float32),
        compiler_params=pltpu.CompilerParams(use_tc_tiling_on_sc=False),
        scratch_types=(
            [pltpu.VMEM((C,), jnp.int32) for _ in range(_NBUF)]
            + [pltpu.VMEM((C, _D), jnp.float32) for _ in range(_NBUF)]
            + [pltpu.SemaphoreType.DMA for _ in range(3 * _NBUF)]
        ),
    )
    def gather_kernel(idx_hbm, table_hbm, out_hbm, *scratch):
        idx_v = scratch[:_NBUF]
        rows_v = scratch[_NBUF : 2 * _NBUF]
        sem_idx = scratch[2 * _NBUF : 3 * _NBUF]
        sem_g = scratch[3 * _NBUF : 4 * _NBUF]
        sem_o = scratch[4 * _NBUF : 5 * _NBUF]

        wid = lax.axis_index("s") * nc + lax.axis_index("c")
        base = wid * b_per_w

        def idx_copy(i, b):
            return pltpu.make_async_copy(
                idx_hbm.at[pl.ds(base + i * C, C)], idx_v[b], sem_idx[b]
            )

        def out_copy(i, b):
            return pltpu.make_async_copy(
                rows_v[b], out_hbm.at[pl.ds(base + i * C, C)], sem_o[b]
            )

        for b in range(_NBUF):
            idx_copy(b, b).start()

        def body(j, carry):
            for b in range(_NBUF):
                i = j + b

                @pl.when(i >= _NBUF)
                def _():
                    out_copy(i - _NBUF, b).wait()

                idx_copy(i, b).wait()
                pltpu.make_async_copy(
                    table_hbm.at[idx_v[b]], rows_v[b], sem_g[b]
                ).start()

            for b in range(_NBUF):
                i = j + b
                pltpu.make_async_copy(
                    table_hbm.at[idx_v[b]], rows_v[b], sem_g[b]
                ).wait()

                @pl.when(i + _NBUF < n_chunks)
                def _():
                    idx_copy(i + _NBUF, b).start()

                out_copy(i, b).start()
            return carry

        lax.fori_loop(0, n_chunks // _NBUF, lambda j, c: body(j * _NBUF, c), 0)

        for b in range(_NBUF):
            out_copy(n_chunks - _NBUF + b, b).wait()

    return gather_kernel


def kernel(token_ids, weight):
    s0, s1 = token_ids.shape
    B = s0 * s1
    idx = token_ids.reshape(B).astype(jnp.int32)
    out = _make_gather(B, 1600)(idx, weight)
    return out.reshape(s0, s1, _D)


# speedup vs baseline: 5.0271x; 1.0006x over previous
import functools

import jax
import jax.numpy as jnp
from jax import lax
from jax.experimental import pallas as pl
from jax.experimental.pallas import tpu as pltpu
from jax.experimental.pallas import tpu_sc as plsc

_D = 32
_NBUF = 2
_R = 8


@functools.cache
def _make_gather(S0: int, S1: int):
    info = plsc.get_sparse_core_info()
    nc, ns = info.num_cores, info.num_subcores
    nw = nc * ns
    rows_per_w = S0 // nw
    n_chunks = rows_per_w // _R
    assert S0 % nw == 0 and rows_per_w % _R == 0 and n_chunks % _NBUF == 0

    mesh = plsc.VectorSubcoreMesh(core_axis_name="c", subcore_axis_name="s")

    @functools.partial(
        pl.kernel,
        mesh=mesh,
        out_type=jax.ShapeDtypeStruct((S0, S1, _D), jnp.float32),
        compiler_params=pltpu.CompilerParams(use_tc_tiling_on_sc=False),
        scratch_types=(
            [pltpu.VMEM((_R, S1), jnp.int32) for _ in range(_NBUF)]
            + [pltpu.VMEM((_R, S1, _D), jnp.float32) for _ in range(_NBUF)]
            + [pltpu.SemaphoreType.DMA for _ in range(3 * _NBUF)]
        ),
    )
    def gather_kernel(idx_hbm, table_hbm, out_hbm, *scratch):
        idx_v = scratch[:_NBUF]
        rows_v = scratch[_NBUF : 2 * _NBUF]
        sem_idx = scratch[2 * _NBUF : 3 * _NBUF]
        sem_g = scratch[3 * _NBUF : 4 * _NBUF]
        sem_o = scratch[4 * _NBUF : 5 * _NBUF]

        wid = lax.axis_index("s") * nc + lax.axis_index("c")
        base = wid * rows_per_w

        def idx_copy(i, b):
            return pltpu.make_async_copy(
                idx_hbm.at[pl.ds(base + i * _R, _R)], idx_v[b], sem_idx[b]
            )

        def out_copy(i, b):
            return pltpu.make_async_copy(
                rows_v[b], out_hbm.at[pl.ds(base + i * _R, _R)], sem_o[b]
            )

        for b in range(_NBUF):
            idx_copy(b, b).start()

        def gather_copies(b):
            return [
                pltpu.make_async_copy(
                    table_hbm.at[idx_v[b].at[r]], rows_v[b].at[r], sem_g[b]
                )
                for r in range(_R)
            ]

        def body(j, carry):
            for b in range(_NBUF):
                i = j + b

                @pl.when(i >= _NBUF)
                def _():
                    out_copy(i - _NBUF, b).wait()

                idx_copy(i, b).wait()
                for cp in gather_copies(b):
                    cp.start()

            for b in range(_NBUF):
                i = j + b
                for cp in gather_copies(b):
                    cp.wait()

                @pl.when(i + _NBUF < n_chunks)
                def _():
                    idx_copy(i + _NBUF, b).start()

                out_copy(i, b).start()
            return carry

        lax.fori_loop(0, n_chunks // _NBUF, lambda j, c: body(j * _NBUF, c), 0)

        for b in range(_NBUF):
            out_copy(n_chunks - _NBUF + b, b).wait()

    return gather_kernel


def kernel(token_ids, weight):
    s0, s1 = token_ids.shape
    return _make_gather(s0, s1)(token_ids, weight)
